# baseline (speedup 1.0000x reference)
import functools

import jax
import jax.numpy as jnp
from jax.experimental import pallas as pl
from jax.experimental.pallas import tpu as pltpu


def _se_step(x_ref, w1_ref, w2_ref, o_ref, *, inv_hw):
    xb = x_ref[...]
    pooled = jnp.sum(xb, axis=-1) * inv_hw
    h = jax.lax.dot_general(pooled, w1_ref[...], (((1,), (1,)), ((), ())),
                            preferred_element_type=jnp.float32)
    h = jnp.maximum(h, 0.0)
    logits = jax.lax.dot_general(h, w2_ref[...], (((1,), (1,)), ((), ())),
                                 preferred_element_type=jnp.float32)
    gate = jax.nn.sigmoid(logits)
    o_ref[...] = xb * gate[:, :, None]


def kernel(x, w1, w2):
    B, C, H, W = x.shape
    HW = H * W
    x3 = x.reshape(B, C, HW)
    TB = 4
    out = pl.pallas_call(
        functools.partial(_se_step, inv_hw=1.0 / float(HW)),
        out_shape=jax.ShapeDtypeStruct((B, C, HW), x.dtype),
        grid=(B // TB,),
        in_specs=[
            pl.BlockSpec((TB, C, HW), lambda b: (b, 0, 0)),
            pl.BlockSpec(w1.shape, lambda b: (0, 0)),
            pl.BlockSpec(w2.shape, lambda b: (0, 0)),
        ],
        out_specs=pl.BlockSpec((TB, C, HW), lambda b: (b, 0, 0)),
        compiler_params=pltpu.CompilerParams(
            dimension_semantics=("parallel",),
            vmem_limit_bytes=10 << 20,
        ),
    )(x3, w1, w2)
    one = w1[0, 0] * 0.0 + 1.0
    return (out * one).reshape(B, C, H, W)


# speedup vs baseline: 1.3342x; 1.3342x over previous
import functools

import jax
import jax.numpy as jnp
from jax.experimental import pallas as pl
from jax.experimental.pallas import tpu as pltpu


def _se_pipe(x_hbm, w1_vmem, w2_vmem, o_hbm, ibuf, obuf, isem, osem,
             *, tb, n_blocks, ni, no, inv_hw):
    core = pl.program_id(0)
    base = core * n_blocks

    def start_in(blk, slot):
        pltpu.make_async_copy(
            x_hbm.at[pl.ds((base + blk) * tb, tb)], ibuf.at[slot], isem.at[slot]
        ).start()

    def wait_in(slot):
        pltpu.make_async_copy(
            x_hbm.at[pl.ds(0, tb)], ibuf.at[slot], isem.at[slot]
        ).wait()

    def start_out(blk, slot):
        pltpu.make_async_copy(
            obuf.at[slot], o_hbm.at[pl.ds((base + blk) * tb, tb)], osem.at[slot]
        ).start()

    def wait_out(slot):
        pltpu.make_async_copy(
            obuf.at[slot], o_hbm.at[pl.ds(0, tb)], osem.at[slot]
        ).wait()

    for k in range(ni):
        start_in(k, k)

    def body(i, _):
        si = jax.lax.rem(i, ni)
        so = jax.lax.rem(i, no)

        @pl.when(i >= no)
        def _():
            wait_out(so)

        wait_in(si)
        xb = ibuf[si]
        pooled = jnp.sum(xb, axis=-1) * inv_hw
        h = jax.lax.dot_general(pooled, w1_vmem[...], (((1,), (1,)), ((), ())),
                                preferred_element_type=jnp.float32)
        h = jnp.maximum(h, 0.0)
        logits = jax.lax.dot_general(h, w2_vmem[...], (((1,), (1,)), ((), ())),
                                     preferred_element_type=jnp.float32)
        gate = jax.nn.sigmoid(logits)
        obuf[so] = xb * gate[:, :, None]

        @pl.when(i + ni < n_blocks)
        def _():
            start_in(i + ni, si)

        start_out(i, so)
        return ()

    jax.lax.fori_loop(0, n_blocks, body, ())
    for k in range(no):
        wait_out((n_blocks - no + k) % no)


def kernel(x, w1, w2):
    B, C, H, W = x.shape
    HW = H * W
    x3 = x.reshape(B, C, HW)
    TB = 4
    NI = NO = 4
    n_blocks = B // TB // 2

    out = pl.pallas_call(
        functools.partial(_se_pipe, tb=TB, n_blocks=n_blocks, ni=NI, no=NO,
                          inv_hw=1.0 / float(HW)),
        out_shape=jax.ShapeDtypeStruct((B, C, HW), x.dtype),
        grid=(2,),
        in_specs=[
            pl.BlockSpec(memory_space=pl.ANY),
            pl.BlockSpec(memory_space=pltpu.VMEM),
            pl.BlockSpec(memory_space=pltpu.VMEM),
        ],
        out_specs=pl.BlockSpec(memory_space=pl.ANY),
        scratch_shapes=[
            pltpu.VMEM((NI, TB, C, HW), jnp.float32),
            pltpu.VMEM((NO, TB, C, HW), jnp.float32),
            pltpu.SemaphoreType.DMA((NI,)),
            pltpu.SemaphoreType.DMA((NO,)),
        ],
        compiler_params=pltpu.CompilerParams(
            dimension_semantics=("parallel",),
            vmem_limit_bytes=32 << 20,
        ),
    )(x3, w1, w2)
    return out.reshape(B, C, H, W)


# speedup vs baseline: 1.3344x; 1.0001x over previous
import functools

import jax
import jax.numpy as jnp
from jax.experimental import pallas as pl
from jax.experimental.pallas import tpu as pltpu


def _se_pipe(x_hbm, w1_vmem, w2_vmem, o_hbm, ibuf, obuf, isem, osem,
             *, tb, n_blocks, ni, no, inv_hw):
    core = pl.program_id(0)
    base = core * n_blocks

    def start_in(blk, slot):
        pltpu.make_async_copy(
            x_hbm.at[pl.ds((base + blk) * tb, tb)], ibuf.at[slot], isem.at[slot]
        ).start()

    def wait_in(slot):
        pltpu.make_async_copy(
            x_hbm.at[pl.ds(0, tb)], ibuf.at[slot], isem.at[slot]
        ).wait()

    def start_out(blk, slot):
        pltpu.make_async_copy(
            obuf.at[slot], o_hbm.at[pl.ds((base + blk) * tb, tb)], osem.at[slot]
        ).start()

    def wait_out(slot):
        pltpu.make_async_copy(
            obuf.at[slot], o_hbm.at[pl.ds(0, tb)], osem.at[slot]
        ).wait()

    for k in range(ni):
        start_in(k, k)

    def body(i, _):
        si = jax.lax.rem(i, ni)
        so = jax.lax.rem(i, no)

        @pl.when(i >= no)
        def _():
            wait_out(so)

        wait_in(si)
        xb = ibuf[si]
        pooled = jnp.sum(xb, axis=-1) * inv_hw
        h = jax.lax.dot_general(pooled, w1_vmem[...], (((1,), (1,)), ((), ())),
                                preferred_element_type=jnp.float32)
        h = jnp.maximum(h, 0.0)
        logits = jax.lax.dot_general(h, w2_vmem[...], (((1,), (1,)), ((), ())),
                                     preferred_element_type=jnp.float32)
        gate = jax.nn.sigmoid(logits)
        obuf[so] = xb * gate[:, :, None]

        @pl.when(i + ni < n_blocks)
        def _():
            start_in(i + ni, si)

        start_out(i, so)
        return ()

    jax.lax.fori_loop(0, n_blocks, body, ())
    for k in range(no):
        wait_out((n_blocks - no + k) % no)


def kernel(x, w1, w2):
    B, C, H, W = x.shape
    HW = H * W
    x3 = x.reshape(B, C, HW)
    TB = 8
    NI = NO = 3
    n_blocks = B // TB // 2

    out = pl.pallas_call(
        functools.partial(_se_pipe, tb=TB, n_blocks=n_blocks, ni=NI, no=NO,
                          inv_hw=1.0 / float(HW)),
        out_shape=jax.ShapeDtypeStruct((B, C, HW), x.dtype),
        grid=(2,),
        in_specs=[
            pl.BlockSpec(memory_space=pl.ANY),
            pl.BlockSpec(memory_space=pltpu.VMEM),
            pl.BlockSpec(memory_space=pltpu.VMEM),
        ],
        out_specs=pl.BlockSpec(memory_space=pl.ANY),
        scratch_shapes=[
            pltpu.VMEM((NI, TB, C, HW), jnp.float32),
            pltpu.VMEM((NO, TB, C, HW), jnp.float32),
            pltpu.SemaphoreType.DMA((NI,)),
            pltpu.SemaphoreType.DMA((NO,)),
        ],
        compiler_params=pltpu.CompilerParams(
            dimension_semantics=("parallel",),
            vmem_limit_bytes=32 << 20,
        ),
    )(x3, w1, w2)
    return out.reshape(B, C, H, W)


# speedup vs baseline: 1.3497x; 1.0115x over previous
import functools

import jax
import jax.numpy as jnp
from jax.experimental import pallas as pl
from jax.experimental.pallas import tpu as pltpu


def _se_pipe(x_hbm, w1_vmem, w2_vmem, o_hbm, ibuf, obuf, isem, osem,
             *, tb, n_blocks, ni, no, inv_hw):
    core = pl.program_id(0)
    base = core * n_blocks

    def start_in(blk, slot):
        pltpu.make_async_copy(
            x_hbm.at[pl.ds((base + blk) * tb, tb)], ibuf.at[slot], isem.at[slot]
        ).start()

    def wait_in(slot):
        pltpu.make_async_copy(
            x_hbm.at[pl.ds(0, tb)], ibuf.at[slot], isem.at[slot]
        ).wait()

    def start_out(blk, slot):
        pltpu.make_async_copy(
            obuf.at[slot], o_hbm.at[pl.ds((base + blk) * tb, tb)], osem.at[slot]
        ).start()

    def wait_out(slot):
        pltpu.make_async_copy(
            obuf.at[slot], o_hbm.at[pl.ds(0, tb)], osem.at[slot]
        ).wait()

    for k in range(ni):
        start_in(k, k)

    def body(i, _):
        si = jax.lax.rem(i, ni)
        so = jax.lax.rem(i, no)

        @pl.when(i >= no)
        def _():
            wait_out(so)

        wait_in(si)
        xb = ibuf[si]
        pooled = jnp.sum(xb, axis=-1) * inv_hw
        h = jax.lax.dot_general(pooled, w1_vmem[...], (((1,), (1,)), ((), ())),
                                preferred_element_type=jnp.float32)
        h = jnp.maximum(h, 0.0)
        logits = jax.lax.dot_general(h, w2_vmem[...], (((1,), (1,)), ((), ())),
                                     preferred_element_type=jnp.float32)
        gate = jax.nn.sigmoid(logits)
        obuf[so] = xb * gate[:, :, None]

        @pl.when(i + ni < n_blocks)
        def _():
            start_in(i + ni, si)

        start_out(i, so)
        return ()

    jax.lax.fori_loop(0, n_blocks, body, ())
    for k in range(no):
        wait_out((n_blocks - no + k) % no)


def kernel(x, w1, w2):
    B, C, H, W = x.shape
    HW = H * W
    x3 = x.reshape(B, C, HW)
    TB = 4
    NI = NO = 6
    n_blocks = B // TB // 2

    out = pl.pallas_call(
        functools.partial(_se_pipe, tb=TB, n_blocks=n_blocks, ni=NI, no=NO,
                          inv_hw=1.0 / float(HW)),
        out_shape=jax.ShapeDtypeStruct((B, C, HW), x.dtype),
        grid=(2,),
        in_specs=[
            pl.BlockSpec(memory_space=pl.ANY),
            pl.BlockSpec(memory_space=pltpu.VMEM),
            pl.BlockSpec(memory_space=pltpu.VMEM),
        ],
        out_specs=pl.BlockSpec(memory_space=pl.ANY),
        scratch_shapes=[
            pltpu.VMEM((NI, TB, C, HW), jnp.float32),
            pltpu.VMEM((NO, TB, C, HW), jnp.float32),
            pltpu.SemaphoreType.DMA((NI,)),
            pltpu.SemaphoreType.DMA((NO,)),
        ],
        compiler_params=pltpu.CompilerParams(
            dimension_semantics=("parallel",),
            vmem_limit_bytes=32 << 20,
        ),
    )(x3, w1, w2)
    return out.reshape(B, C, H, W)
